# baseline (speedup 1.0000x reference)
import functools

import jax
import jax.numpy as jnp
from jax import lax
from jax.experimental import pallas as pl
from jax.experimental.pallas import tpu as pltpu
from jax.experimental.pallas import tpu_sc as plsc

_D = 64
_SCALE = float(_D) ** 0.5
_CHUNK = 100
_NBUF = 4


@functools.cache
def _build(batch, seq):
    info = plsc.get_sparse_core_info()
    nc, ns, nl = info.num_cores, info.num_subcores, info.num_lanes
    nw = nc * ns
    n_idx = batch * seq
    assert batch % nw == 0 and seq % _CHUNK == 0
    batches_per_w = batch // nw
    b_per_w = n_idx // nw
    chunks_per_seq = seq // _CHUNK
    n_chunks = batches_per_w * chunks_per_seq
    assert n_chunks % _NBUF == 0 and n_chunks >= 2 * _NBUF

    mesh = plsc.VectorSubcoreMesh(core_axis_name="c", subcore_axis_name="s")

    @functools.partial(
        pl.kernel,
        mesh=mesh,
        compiler_params=pltpu.CompilerParams(use_tc_tiling_on_sc=False),
        out_type=jax.ShapeDtypeStruct((batch, seq, _D), jnp.float32),
        scratch_types=[
            pltpu.VMEM((n_chunks, _CHUNK), jnp.int32),
            pltpu.VMEM((_NBUF, _CHUNK, _D), jnp.float32),
            pltpu.VMEM((_NBUF, _CHUNK, _D), jnp.float32),
        ]
        + [pltpu.SemaphoreType.DMA] * (2 * _NBUF),
    )
    def emb_kernel(idx_hbm, table_hbm, out_hbm, idx_v, in_bufs, out_bufs, *sems):
        sin, sout = sems[:_NBUF], sems[_NBUF:]
        wid = lax.axis_index("s") * nc + lax.axis_index("c")
        base_batch = wid * batches_per_w
        pltpu.sync_copy(idx_hbm.at[wid], idx_v)

        def gather_copy(g, b):
            return pltpu.make_async_copy(
                table_hbm.at[idx_v.at[g]], in_bufs.at[b], sin[b]
            )

        def out_copy(g, b):
            bb = base_batch + g // chunks_per_seq
            off = (g % chunks_per_seq) * _CHUNK
            return pltpu.make_async_copy(
                out_bufs.at[b], out_hbm.at[bb, pl.ds(off, _CHUNK)], sout[b]
            )

        def scale(b):
            def body(r4, c):
                for dr in range(4):
                    r = r4 * 4 + dr
                    for j in range(_D // nl):
                        sl = pl.ds(j * nl, nl)
                        out_bufs[b, r, sl] = in_bufs[b, r, sl] * _SCALE
                return c

            lax.fori_loop(0, _CHUNK // 4, body, 0)

        for b in range(_NBUF):
            gather_copy(b, b).start()

        for g in range(_NBUF):
            b = g
            gather_copy(g, b).wait()
            scale(b)
            out_copy(g, b).start()
            gather_copy(g + _NBUF, b).start()

        def mid(i, c):
            for b in range(_NBUF):
                g = i * _NBUF + _NBUF + b
                gather_copy(g, b).wait()
                out_copy(g - _NBUF, b).wait()
                scale(b)
                out_copy(g, b).start()
                gather_copy(g + _NBUF, b).start()
            return c

        lax.fori_loop(0, (n_chunks - 2 * _NBUF) // _NBUF, mid, 0)

        for k in range(_NBUF):
            g = n_chunks - _NBUF + k
            gather_copy(g, k).wait()
            out_copy(g - _NBUF, k).wait()
            scale(k)
            out_copy(g, k).start()
        for k in range(_NBUF):
            out_copy(n_chunks - _NBUF + k, k).wait()

    def run(x, tab):
        idx = x.reshape(nw, n_chunks, _CHUNK).astype(jnp.int32)
        return emb_kernel(idx, tab)

    return run


def kernel(x, table):
    b, s = x.shape
    return _build(b, s)(x, table)


# speedup vs baseline: 1.2201x; 1.2201x over previous
import functools

import jax
import jax.numpy as jnp
from jax import lax
from jax.experimental import pallas as pl
from jax.experimental.pallas import tpu as pltpu
from jax.experimental.pallas import tpu_sc as plsc

_D = 64
_DP = 128
_SCALE = float(_D) ** 0.5
_CHUNK = 128
_NBUF = 2


@functools.cache
def _build(n_idx, vocab):
    info = plsc.get_sparse_core_info()
    nc, ns, nl = info.num_cores, info.num_subcores, info.num_lanes
    nw = nc * ns
    assert n_idx % (nw * _CHUNK) == 0
    b_per_w = n_idx // nw
    n_chunks = b_per_w // _CHUNK
    assert n_chunks % _NBUF == 0 and n_chunks >= 2 * _NBUF

    mesh = plsc.VectorSubcoreMesh(core_axis_name="c", subcore_axis_name="s")

    @functools.partial(
        pl.kernel,
        mesh=mesh,
        out_type=jax.ShapeDtypeStruct((n_idx, _D), jnp.float32),
        scratch_types=[
            pltpu.VMEM((b_per_w,), jnp.int32),
            pltpu.VMEM((_NBUF, _CHUNK, _DP), jnp.float32),
            pltpu.VMEM((_NBUF, _CHUNK, _D), jnp.float32),
        ]
        + [pltpu.SemaphoreType.DMA] * (2 * _NBUF),
    )
    def emb_kernel(idx_hbm, table_hbm, out_hbm, idx_v, in_bufs, out_bufs, *sems):
        sin, sout = sems[:_NBUF], sems[_NBUF:]
        wid = lax.axis_index("s") * nc + lax.axis_index("c")
        base = wid * b_per_w
        pltpu.sync_copy(idx_hbm.at[pl.ds(base, b_per_w)], idx_v)

        def gather_copy(g, b):
            return pltpu.make_async_copy(
                table_hbm.at[idx_v.at[pl.ds(g * _CHUNK, _CHUNK)]],
                in_bufs.at[b],
                sin[b],
            )

        def out_copy(g, b):
            return pltpu.make_async_copy(
                out_bufs.at[b], out_hbm.at[pl.ds(base + g * _CHUNK, _CHUNK)], sout[b]
            )

        def scale(b):
            def body(r4, c):
                for dr in range(4):
                    r = r4 * 4 + dr
                    for j in range(_D // nl):
                        sl = pl.ds(j * nl, nl)
                        out_bufs[b, r, sl] = in_bufs[b, r, sl] * _SCALE
                return c

            lax.fori_loop(0, _CHUNK // 4, body, 0)

        for b in range(_NBUF):
            gather_copy(b, b).start()

        for g in range(_NBUF):
            b = g
            gather_copy(g, b).wait()
            scale(b)
            out_copy(g, b).start()
            gather_copy(g + _NBUF, b).start()

        def mid(i, c):
            for b in range(_NBUF):
                g = i * _NBUF + _NBUF + b
                gather_copy(g, b).wait()
                out_copy(g - _NBUF, b).wait()
                scale(b)
                out_copy(g, b).start()
                gather_copy(g + _NBUF, b).start()
            return c

        lax.fori_loop(0, (n_chunks - 2 * _NBUF) // _NBUF, mid, 0)

        for k in range(_NBUF):
            g = n_chunks - _NBUF + k
            gather_copy(g, k).wait()
            out_copy(g - _NBUF, k).wait()
            scale(k)
            out_copy(g, k).start()
        for k in range(_NBUF):
            out_copy(n_chunks - _NBUF + k, k).wait()

    return emb_kernel


def kernel(x, table):
    b, s = x.shape
    v, _ = table.shape
    idx = x.reshape(b * s).astype(jnp.int32)
    padded = jnp.pad(table, ((0, 0), (0, _DP - _D)))
    out = _build(b * s, v)(idx, padded)
    return out.reshape(b, s, _D)
